# baseline (speedup 1.0000x reference)
import jax
import jax.numpy as jnp
from jax.experimental import pallas as pl
from jax.experimental.pallas import tpu as pltpu

_BN = 256


def _cluster_kernel(data_ref, cent_ref, out_ref):
    data = data_ref[:, :]
    cent = cent_ref[:, :]
    xx = jnp.sum(data * data, axis=1)
    cc = jnp.sum(cent * cent, axis=1)
    g = jax.lax.dot_general(
        cent, data, (((1,), (1,)), ((), ())),
        preferred_element_type=jnp.float32)
    dist = cc[:, None] + xx[None, :] - 2.0 * g
    q = 1.0 / (1.0 + dist)
    q = q * q * 0.5
    s = jnp.sum(q, axis=0)
    out_ref[:, :] = q / s[None, :]


def kernel(data, centroids):
    n, d = data.shape
    k, _ = centroids.shape
    return pl.pallas_call(
        _cluster_kernel,
        grid=(n // _BN,),
        in_specs=[
            pl.BlockSpec((_BN, d), lambda i: (i, 0)),
            pl.BlockSpec((k, d), lambda i: (0, 0)),
        ],
        out_specs=pl.BlockSpec((k, _BN), lambda i: (0, i)),
        out_shape=jax.ShapeDtypeStruct((k, n), jnp.float32),
        compiler_params=pltpu.CompilerParams(
            dimension_semantics=("parallel",)),
    )(data, centroids)


# speedup vs baseline: 1.2748x; 1.2748x over previous
import jax
import jax.numpy as jnp
from jax.experimental import pallas as pl


def _cluster_kernel(data_ref, cent_ref, out_ref):
    data = data_ref[:, :]
    cent = cent_ref[:, :]
    xx = jnp.sum(data * data, axis=1)
    ccp1 = jnp.sum(cent * cent, axis=1) + 1.0
    g = jax.lax.dot_general(
        cent * -2.0, data, (((1,), (1,)), ((), ())),
        preferred_element_type=jnp.float32)
    u = g + ccp1[:, None] + xx[None, :]
    r = 1.0 / u
    t = r * r
    s = jnp.sum(t, axis=0)
    out_ref[:, :] = t * (1.0 / s)[None, :]


def kernel(data, centroids):
    n, _ = data.shape
    k, _ = centroids.shape
    return pl.pallas_call(
        _cluster_kernel,
        out_shape=jax.ShapeDtypeStruct((k, n), jnp.float32),
    )(data, centroids)


# speedup vs baseline: 1.3515x; 1.0602x over previous
import jax
import jax.numpy as jnp
from jax.experimental import pallas as pl


def _cluster_kernel(data_ref, cent_ref, out_ref):
    data = data_ref[:, :]
    cent = cent_ref[:, :]
    xx = jnp.sum(data * data, axis=1)
    ccp1 = jnp.sum(cent * cent, axis=1) + 1.0
    g = jax.lax.dot_general(
        cent * -2.0, data, (((1,), (1,)), ((), ())),
        preferred_element_type=jnp.float32)
    u = g + ccp1[:, None] + xx[None, :]
    r = 1.0 / u
    t = r * r
    s = jnp.sum(t, axis=0)
    out_ref[:, :] = t * (1.0 / s)[None, :]


_BN = 1024


def kernel(data, centroids):
    n, d = data.shape
    k, _ = centroids.shape
    return pl.pallas_call(
        _cluster_kernel,
        grid=(n // _BN,),
        in_specs=[
            pl.BlockSpec((_BN, d), lambda i: (i, 0)),
            pl.BlockSpec((k, d), lambda i: (0, 0)),
        ],
        out_specs=pl.BlockSpec((k, _BN), lambda i: (0, i)),
        out_shape=jax.ShapeDtypeStruct((k, n), jnp.float32),
    )(data, centroids)
